# baseline (speedup 1.0000x reference)
import functools

import jax
import jax.numpy as jnp
from jax import lax
from jax.experimental import pallas as pl
from jax.experimental.pallas import tpu as pltpu
from jax.experimental.pallas import tpu_sc as plsc

VOCAB = 1000000
H = 64
S = 200
B = 4096
T = 4
L = 16
NC = 2
NS = 16
NW = NC * NS
BPW = B // NW
NCHUNK = BPW // L


def _sc_body(seqs_hbm, tgt_hbm, pairs_hbm, w_hbm, out_hbm,
             seqs_l, tgt_v, tok_v, pair_v, tiles_v, w_v, wbc_v, pred_v, sem):
    wid = lax.axis_index("s") * NC + lax.axis_index("c")
    base = wid * BPW

    pltpu.sync_copy(seqs_hbm.at[:, pl.ds(base, BPW)], seqs_l)
    pltpu.sync_copy(tgt_hbm.at[:, pl.ds(base, BPW)], tgt_v)
    pltpu.sync_copy(w_hbm, w_v)

    iidx = [lax.iota(jnp.int32, L) + c * L for c in range(NCHUNK)]
    for t in range(T):
        for c in range(NCHUNK):
            sl = pl.ds(c * L, L)
            tok = plsc.load_gather(seqs_l, [tgt_v[t, sl], iidx[c]])
            tok_v[t, sl] = tok
            pair_v[t, sl] = lax.shift_right_logical(tok, 1)

    def wfill(j, carry):
        wbc_v[j, :] = plsc.load_gather(w_v, [jnp.full((L,), j, jnp.int32)])
        return carry
    lax.fori_loop(0, T * H, wfill, 0)

    cps = [pltpu.async_copy(pairs_hbm.at[pair_v.at[t]],
                            tiles_v.at[pl.ds(t * BPW, BPW)], sem)
           for t in range(T)]
    for cp in cps:
        cp.wait()

    for t in range(T):
        kidx = [lax.iota(jnp.int32, L) + (t * BPW + c * L)
                for c in range(NCHUNK)]
        parcol = [(tok_v[t, pl.ds(c * L, L)] & 1) * H for c in range(NCHUNK)]

        def hbody(h, accs, t=t, kidx=kidx, parcol=parcol):
            bw = wbc_v[t * H + h, :]
            return tuple(
                accs[c] + plsc.load_gather(tiles_v, [kidx[c], parcol[c] + h])
                * bw
                for c in range(NCHUNK))

        accs = lax.fori_loop(
            0, H, hbody, tuple(jnp.zeros((L,), jnp.float32)
                               for _ in range(NCHUNK)))
        for c in range(NCHUNK):
            sl = pl.ds(c * L, L)
            if t == 0:
                pred_v[sl] = accs[c]
            else:
                pred_v[sl] = pred_v[sl] + accs[c]

    pltpu.sync_copy(pred_v, out_hbm.at[pl.ds(base, BPW)])


_sc_gather = functools.partial(
    pl.kernel,
    out_type=jax.ShapeDtypeStruct((B,), jnp.float32),
    mesh=plsc.VectorSubcoreMesh(core_axis_name="c", subcore_axis_name="s"),
    compiler_params=pltpu.CompilerParams(needs_layout_passes=False),
    scratch_types=[
        pltpu.VMEM((S, BPW), jnp.int32),
        pltpu.VMEM((T, BPW), jnp.int32),
        pltpu.VMEM((T, BPW), jnp.int32),
        pltpu.VMEM((T, BPW), jnp.int32),
        pltpu.VMEM((T * BPW, 2 * H), jnp.float32),
        pltpu.VMEM((T * H,), jnp.float32),
        pltpu.VMEM((T * H, L), jnp.float32),
        pltpu.VMEM((BPW,), jnp.float32),
        pltpu.SemaphoreType.DMA,
    ],
)(_sc_body)


def _loss_body(pred_ref, lab_ref, b_ref, out_ref):
    p = pred_ref[:] + b_ref[0]
    lab = lab_ref[:]
    terms = (jnp.maximum(p, 0.0) - p * lab
             + jnp.log(1.0 + jnp.exp(-jnp.abs(p))))
    out_ref[0, 0] = jnp.sum(terms) * (1.0 / B)


_loss_call = pl.pallas_call(
    _loss_body,
    out_shape=jax.ShapeDtypeStruct((1, 1), jnp.float32),
    in_specs=[
        pl.BlockSpec(memory_space=pltpu.VMEM),
        pl.BlockSpec(memory_space=pltpu.VMEM),
        pl.BlockSpec(memory_space=pltpu.SMEM),
    ],
    out_specs=pl.BlockSpec(memory_space=pltpu.SMEM),
)


def kernel(seqs, targets, labels, table, W, b):
    seqs32 = seqs.astype(jnp.int32)
    pairs = table.reshape(VOCAB // 2, 2 * H)
    w_flat = W.reshape(-1)
    pred = _sc_gather(seqs32, targets, pairs, w_flat)
    loss = _loss_call(pred.reshape(B // 128, 128),
                      labels.reshape(B // 128, 128), b)
    return loss[0, 0]


# speedup vs baseline: 1.0031x; 1.0031x over previous
import functools

import jax
import jax.numpy as jnp
from jax import lax
from jax.experimental import pallas as pl
from jax.experimental.pallas import tpu as pltpu
from jax.experimental.pallas import tpu_sc as plsc

VOCAB = 1000000
H = 64
S = 200
B = 4096
T = 4
L = 16
NC = 2
NS = 16
NW = NC * NS
BPW = B // NW
NCHUNK = BPW // L


def _sc_body(seqs_hbm, tgt_hbm, table_hbm, w_hbm, out_hbm,
             seqs_l, tgt_v, tok_v, rows_v, w_v, wbc_v, pred_v, sem):
    wid = lax.axis_index("s") * NC + lax.axis_index("c")
    base = wid * BPW

    pltpu.sync_copy(seqs_hbm.at[:, pl.ds(base, BPW)], seqs_l)
    pltpu.sync_copy(tgt_hbm.at[:, pl.ds(base, BPW)], tgt_v)
    pltpu.sync_copy(w_hbm, w_v)

    iidx = [lax.iota(jnp.int32, L) + c * L for c in range(NCHUNK)]
    for t in range(T):
        for c in range(NCHUNK):
            sl = pl.ds(c * L, L)
            tok_v[t, sl] = plsc.load_gather(seqs_l, [tgt_v[t, sl], iidx[c]])

    def wfill(j, carry):
        wbc_v[j, :] = plsc.load_gather(w_v, [jnp.full((L,), j, jnp.int32)])
        return carry
    lax.fori_loop(0, T * H, wfill, 0)

    cps = [pltpu.async_copy(table_hbm.at[tok_v.at[t]],
                            rows_v.at[pl.ds(t * BPW, BPW)], sem)
           for t in range(T)]
    for cp in cps:
        cp.wait()

    for t in range(T):
        kidx = [lax.iota(jnp.int32, L) + (t * BPW + c * L)
                for c in range(NCHUNK)]

        def hbody(h, accs, t=t, kidx=kidx):
            bw = wbc_v[t * H + h, :]
            colh = jnp.full((L,), h, jnp.int32)
            return tuple(
                accs[c] + plsc.load_gather(rows_v, [kidx[c], colh]) * bw
                for c in range(NCHUNK))

        accs = lax.fori_loop(
            0, H, hbody, tuple(jnp.zeros((L,), jnp.float32)
                               for _ in range(NCHUNK)))
        for c in range(NCHUNK):
            sl = pl.ds(c * L, L)
            if t == 0:
                pred_v[sl] = accs[c]
            else:
                pred_v[sl] = pred_v[sl] + accs[c]

    pltpu.sync_copy(pred_v, out_hbm.at[pl.ds(base, BPW)])


_sc_gather = functools.partial(
    pl.kernel,
    out_type=jax.ShapeDtypeStruct((B,), jnp.float32),
    mesh=plsc.VectorSubcoreMesh(core_axis_name="c", subcore_axis_name="s"),
    compiler_params=pltpu.CompilerParams(needs_layout_passes=False,
                                         use_tc_tiling_on_sc=False),
    scratch_types=[
        pltpu.VMEM((S, BPW), jnp.int32),
        pltpu.VMEM((T, BPW), jnp.int32),
        pltpu.VMEM((T, BPW), jnp.int32),
        pltpu.VMEM((T * BPW, H), jnp.float32),
        pltpu.VMEM((T * H,), jnp.float32),
        pltpu.VMEM((T * H, L), jnp.float32),
        pltpu.VMEM((BPW,), jnp.float32),
        pltpu.SemaphoreType.DMA,
    ],
)(_sc_body)


def _loss_body(pred_ref, lab_ref, b_ref, out_ref):
    p = pred_ref[:] + b_ref[0]
    lab = lab_ref[:]
    terms = (jnp.maximum(p, 0.0) - p * lab
             + jnp.log(1.0 + jnp.exp(-jnp.abs(p))))
    out_ref[0, 0] = jnp.sum(terms) * (1.0 / B)


_loss_call = pl.pallas_call(
    _loss_body,
    out_shape=jax.ShapeDtypeStruct((1, 1), jnp.float32),
    in_specs=[
        pl.BlockSpec(memory_space=pltpu.VMEM),
        pl.BlockSpec(memory_space=pltpu.VMEM),
        pl.BlockSpec(memory_space=pltpu.SMEM),
    ],
    out_specs=pl.BlockSpec(memory_space=pltpu.SMEM),
)


def kernel(seqs, targets, labels, table, W, b):
    seqs32 = seqs.astype(jnp.int32)
    w_flat = W.reshape(-1)
    pred = _sc_gather(seqs32, targets, table, w_flat)
    loss = _loss_call(pred.reshape(B // 128, 128),
                      labels.reshape(B // 128, 128), b)
    return loss[0, 0]
